# Initial kernel scaffold; baseline (speedup 1.0000x reference)
import jax
import jax.numpy as jnp
from jax.experimental import pallas as pl


def kernel(x, edge_index, W):
    raise NotImplementedError("write your pallas kernel here")


# speedup vs baseline: 5.9034x; 5.9034x over previous
import functools

import jax
import jax.numpy as jnp
from jax import lax
from jax.experimental import pallas as pl
from jax.experimental.pallas import tpu as pltpu
from jax.experimental.pallas import tpu_sc as plsc

NC, NS, L = 2, 16, 16
NW = NC * NS
CHUNK = 128
TRASH = 16


def _mesh():
    return plsc.VectorSubcoreMesh(
        core_axis_name="c", subcore_axis_name="s",
        num_cores=NC, num_subcores=NS)


@functools.lru_cache(maxsize=None)
def _build(N, E, C):
    EPT = -(-E // (NW * CHUNK * 4)) * CHUNK * 4
    E_pad = NW * EPT
    CPT = EPT // CHUNK
    NBA = -(-(N + TRASH) // (NS * CHUNK)) * NS * CHUNK
    DPT = NBA // NS

    HALF = N // 2
    assert N % 2 == 0 and HALF % 8 == 0
    NBH = HALF + TRASH
    RPH = (HALF // NS) // 8 * 8
    RTH = HALF - RPH * NS
    EPS = E_pad // NS
    CPS = EPS // CHUNK
    WIN = 80
    assert CPS % WIN == 0 and WIN % 4 == 0

    @functools.partial(
        pl.kernel, mesh=_mesh(),
        out_type=jax.ShapeDtypeStruct((NC, NBA), jnp.float32),
        scratch_types=[
            pltpu.VMEM((CPT, CHUNK), jnp.int32),
            pltpu.VMEM((CHUNK,), jnp.float32),
            pltpu.VMEM_SHARED((NBA,), jnp.float32),
            pltpu.SemaphoreType.DMA,
        ],
    )
    def deg_kernel(row3_hbm, out_hbm, idx_v, stage_v, sdeg, sem):
        cid = lax.axis_index("c")
        sid = lax.axis_index("s")
        wid = sid * NC + cid
        pltpu.sync_copy(row3_hbm.at[wid], idx_v)

        def fill(val):
            def fb(i, _):
                stage_v[pl.ds(i * L, L)] = jnp.full((L,), val, jnp.float32)
                return 0
            lax.fori_loop(0, CHUNK // L, fb, 0)

        fill(0.0)
        z0 = sid * DPT
        for j in range(DPT // CHUNK):
            pltpu.sync_copy(stage_v, sdeg.at[pl.ds(z0 + j * CHUNK, CHUNK)])
        plsc.subcore_barrier()
        fill(1.0)

        def hb(g, _):
            c = g * 4
            s0 = pltpu.async_copy(stage_v, sdeg.at[idx_v.at[c]], sem, add=True)
            s1 = pltpu.async_copy(stage_v, sdeg.at[idx_v.at[c + 1]], sem, add=True)
            s2 = pltpu.async_copy(stage_v, sdeg.at[idx_v.at[c + 2]], sem, add=True)
            s3 = pltpu.async_copy(stage_v, sdeg.at[idx_v.at[c + 3]], sem, add=True)
            s0.wait(); s1.wait(); s2.wait(); s3.wait()
            return 0
        lax.fori_loop(0, CPT // 4, hb, 0)

        plsc.subcore_barrier()
        pltpu.sync_copy(sdeg.at[pl.ds(z0, DPT)],
                        out_hbm.at[cid].at[pl.ds(z0, DPT)])

    def _scale_body(ptsT_ref, x_ref, h_ref, dinv_ref):
        deg = jnp.sum(ptsT_ref[...], axis=1, keepdims=True)
        dinv = lax.rsqrt(deg + 1.0)
        dinv_ref[...] = dinv
        h_ref[...] = x_ref[...] * dinv

    scale_call = pl.pallas_call(
        _scale_body,
        out_shape=(jax.ShapeDtypeStruct((N, C), jnp.float32),
                   jax.ShapeDtypeStruct((N, 1), jnp.float32)),
    )

    @functools.partial(
        pl.kernel, mesh=_mesh(),
        out_type=jax.ShapeDtypeStruct((N, C), jnp.float32),
        scratch_types=[
            pltpu.VMEM((WIN, CHUNK), jnp.int32),
            pltpu.VMEM((WIN, CHUNK), jnp.int32),
            pltpu.VMEM((CHUNK, C), jnp.float32),
            pltpu.VMEM((CHUNK, C), jnp.float32),
            pltpu.VMEM((CHUNK, C), jnp.float32),
            pltpu.VMEM((CHUNK, C), jnp.float32),
            pltpu.VMEM_SHARED((NBH, C), jnp.float32),
            pltpu.SemaphoreType.DMA,
            pltpu.SemaphoreType.DMA,
            pltpu.SemaphoreType.DMA,
            pltpu.SemaphoreType.DMA,
            pltpu.SemaphoreType.DMA,
        ],
    )
    def scat_kernel(h_hbm, col3_hbm, row3_hbm, out_hbm,
                    col_v, row_v, b0, b1, b2, b3, acc,
                    g0, g1, g2, g3, ssem):
        cid = lax.axis_index("c")
        sid = lax.axis_index("s")
        base = cid * HALF

        r0 = sid * RPH
        pltpu.sync_copy(h_hbm.at[pl.ds(base + r0, RPH)], acc.at[pl.ds(r0, RPH)])
        if RTH:
            @pl.when(sid == 0)
            def _():
                pltpu.sync_copy(h_hbm.at[pl.ds(base + RPH * NS, RTH)],
                                acc.at[pl.ds(RPH * NS, RTH)])
        plsc.subcore_barrier()

        def gbody(g, _):
            c = g * 4
            d0 = pltpu.async_copy(h_hbm.at[col_v.at[c]], b0, g0)
            d1 = pltpu.async_copy(h_hbm.at[col_v.at[c + 1]], b1, g1)
            d2 = pltpu.async_copy(h_hbm.at[col_v.at[c + 2]], b2, g2)
            d3 = pltpu.async_copy(h_hbm.at[col_v.at[c + 3]], b3, g3)
            d0.wait()
            s0 = pltpu.async_copy(b0, acc.at[row_v.at[c]], ssem, add=True)
            d1.wait()
            s1 = pltpu.async_copy(b1, acc.at[row_v.at[c + 1]], ssem, add=True)
            d2.wait()
            s2 = pltpu.async_copy(b2, acc.at[row_v.at[c + 2]], ssem, add=True)
            d3.wait()
            s3 = pltpu.async_copy(b3, acc.at[row_v.at[c + 3]], ssem, add=True)
            s0.wait(); s1.wait(); s2.wait(); s3.wait()
            return 0

        def tb(k, _):
            ci = k // (CHUNK // L)
            j = k % (CHUNK // L)
            v = row_v[ci, pl.ds(j * L, L)]
            lv = v - base
            oob = (lv < 0) | (lv >= HALF)
            lv = jnp.where(oob, HALF + lax.iota(jnp.int32, L), lv)
            row_v[ci, pl.ds(j * L, L)] = lv
            return 0

        for w in range(CPS // WIN):
            pltpu.sync_copy(col3_hbm.at[sid].at[pl.ds(w * WIN, WIN)], col_v)
            pltpu.sync_copy(row3_hbm.at[sid].at[pl.ds(w * WIN, WIN)], row_v)
            lax.fori_loop(0, WIN * (CHUNK // L), tb, 0)
            lax.fori_loop(0, WIN // 4, gbody, 0)

        plsc.subcore_barrier()
        pltpu.sync_copy(acc.at[pl.ds(r0, RPH)],
                        out_hbm.at[pl.ds(base + r0, RPH)])
        if RTH:
            @pl.when(sid == 0)
            def _():
                pltpu.sync_copy(acc.at[pl.ds(RPH * NS, RTH)],
                                out_hbm.at[pl.ds(base + RPH * NS, RTH)])

    def _final_body(p_ref, dinv_ref, wt_ref, o_ref):
        a = p_ref[...] * dinv_ref[...]
        o_ref[...] = lax.dot_general(
            a, wt_ref[...], (((1,), (0,)), ((), ())),
            preferred_element_type=jnp.float32)

    final_call = pl.pallas_call(
        _final_body,
        out_shape=jax.ShapeDtypeStruct((N, C), jnp.float32),
    )

    return deg_kernel, scale_call, scat_kernel, final_call, E_pad, EPT, EPS


def kernel(x, edge_index, W):
    N, C = x.shape
    E = edge_index.shape[1]
    deg_kernel, scale_call, scat_kernel, final_call, E_pad, EPT, EPS = \
        _build(N, E, C)

    row = edge_index[0]
    col = edge_index[1]
    pad = E_pad - E
    if pad:
        pad_iota = jnp.arange(pad, dtype=jnp.int32)
        row = jnp.concatenate([row, N + (pad_iota % TRASH)])
        col = jnp.concatenate([col, pad_iota % TRASH])
    rowd = row.reshape(NW, EPT // CHUNK, CHUNK)
    rows = row.reshape(NS, EPS // CHUNK, CHUNK)
    cols = col.reshape(NS, EPS // CHUNK, CHUNK)

    parts = deg_kernel(rowd)
    partsT = parts.T[:N]
    h, dinv = scale_call(partsT, x)
    p = scat_kernel(h, cols, rows)
    return final_call(p, dinv, W.T)


# speedup vs baseline: 8.6562x; 1.4663x over previous
import functools

import jax
import jax.numpy as jnp
from jax import lax
from jax.experimental import pallas as pl
from jax.experimental.pallas import tpu as pltpu
from jax.experimental.pallas import tpu_sc as plsc

NC, NS, L = 2, 16, 16
NW = NC * NS
CHUNK = 128
TRASH = 16


def _mesh():
    return plsc.VectorSubcoreMesh(
        core_axis_name="c", subcore_axis_name="s",
        num_cores=NC, num_subcores=NS)


@functools.lru_cache(maxsize=None)
def _build(N, E, C):
    EPT = -(-E // (NW * CHUNK * 4)) * CHUNK * 4
    E_pad = NW * EPT
    CPT = EPT // CHUNK
    NBA = -(-(N + TRASH) // (NS * CHUNK)) * NS * CHUNK
    DPT = NBA // NS

    NBH = N + TRASH
    RPH = (N // NS) // 8 * 8
    RTH = N - RPH * NS
    CPS = EPT // CHUNK
    WIN = 40
    assert CPS % WIN == 0 and WIN % 2 == 0

    @functools.partial(
        pl.kernel, mesh=_mesh(),
        out_type=jax.ShapeDtypeStruct((NC, NBA), jnp.float32),
        scratch_types=[
            pltpu.VMEM((CPT, CHUNK), jnp.int32),
            pltpu.VMEM((CHUNK,), jnp.float32),
            pltpu.VMEM_SHARED((NBA,), jnp.float32),
            pltpu.SemaphoreType.DMA,
        ],
    )
    def deg_kernel(row3_hbm, out_hbm, idx_v, stage_v, sdeg, sem):
        cid = lax.axis_index("c")
        sid = lax.axis_index("s")
        wid = sid * NC + cid
        pltpu.sync_copy(row3_hbm.at[wid], idx_v)

        def fill(val):
            def fb(i, _):
                stage_v[pl.ds(i * L, L)] = jnp.full((L,), val, jnp.float32)
                return 0
            lax.fori_loop(0, CHUNK // L, fb, 0)

        fill(0.0)
        z0 = sid * DPT
        for j in range(DPT // CHUNK):
            pltpu.sync_copy(stage_v, sdeg.at[pl.ds(z0 + j * CHUNK, CHUNK)])
        plsc.subcore_barrier()
        fill(1.0)

        def hb(g, _):
            c = g * 4
            s0 = pltpu.async_copy(stage_v, sdeg.at[idx_v.at[c]], sem, add=True)
            s1 = pltpu.async_copy(stage_v, sdeg.at[idx_v.at[c + 1]], sem, add=True)
            s2 = pltpu.async_copy(stage_v, sdeg.at[idx_v.at[c + 2]], sem, add=True)
            s3 = pltpu.async_copy(stage_v, sdeg.at[idx_v.at[c + 3]], sem, add=True)
            s0.wait(); s1.wait(); s2.wait(); s3.wait()
            return 0
        lax.fori_loop(0, CPT // 4, hb, 0)

        plsc.subcore_barrier()
        pltpu.sync_copy(sdeg.at[pl.ds(z0, DPT)],
                        out_hbm.at[cid].at[pl.ds(z0, DPT)])

    def _scale_body(ptsT_ref, x_ref, h_ref, dinv_ref):
        deg = jnp.sum(ptsT_ref[...], axis=1, keepdims=True)
        dinv = lax.rsqrt(deg + 1.0)
        dinv_ref[...] = dinv
        h_ref[...] = x_ref[...] * dinv

    scale_call = pl.pallas_call(
        _scale_body,
        out_shape=(jax.ShapeDtypeStruct((N, C), jnp.float32),
                   jax.ShapeDtypeStruct((N, 1), jnp.float32)),
    )

    @functools.partial(
        pl.kernel, mesh=_mesh(),
        out_type=jax.ShapeDtypeStruct((NC, N, C), jnp.float32),
        scratch_types=[
            pltpu.VMEM((WIN, CHUNK), jnp.int32),
            pltpu.VMEM((WIN, CHUNK), jnp.int32),
            pltpu.VMEM((CHUNK, C), jnp.float32),
            pltpu.VMEM((CHUNK, C), jnp.float32),
            pltpu.VMEM_SHARED((NBH, C), jnp.float32),
            pltpu.SemaphoreType.DMA,
            pltpu.SemaphoreType.DMA,
            pltpu.SemaphoreType.DMA,
        ],
    )
    def scat_kernel(h_hbm, col3_hbm, row3_hbm, out_hbm,
                    col_v, row_v, b0, b1, acc, g0, g1, ssem):
        cid = lax.axis_index("c")
        sid = lax.axis_index("s")
        wid = sid * NC + cid

        r0 = sid * RPH
        pltpu.sync_copy(h_hbm.at[pl.ds(r0, RPH)], acc.at[pl.ds(r0, RPH)])
        if RTH:
            @pl.when(sid == 0)
            def _():
                pltpu.sync_copy(h_hbm.at[pl.ds(RPH * NS, RTH)],
                                acc.at[pl.ds(RPH * NS, RTH)])
        plsc.subcore_barrier()

        def gbody(g, _):
            c = g * 2
            d0 = pltpu.async_copy(h_hbm.at[col_v.at[c]], b0, g0)
            d1 = pltpu.async_copy(h_hbm.at[col_v.at[c + 1]], b1, g1)
            d0.wait()
            s0 = pltpu.async_copy(b0, acc.at[row_v.at[c]], ssem, add=True)
            d1.wait()
            s1 = pltpu.async_copy(b1, acc.at[row_v.at[c + 1]], ssem, add=True)
            s0.wait(); s1.wait()
            return 0

        for w in range(CPS // WIN):
            pltpu.sync_copy(col3_hbm.at[wid].at[pl.ds(w * WIN, WIN)], col_v)
            pltpu.sync_copy(row3_hbm.at[wid].at[pl.ds(w * WIN, WIN)], row_v)
            lax.fori_loop(0, WIN // 2, gbody, 0)

        plsc.subcore_barrier()
        pltpu.sync_copy(acc.at[pl.ds(r0, RPH)],
                        out_hbm.at[cid].at[pl.ds(r0, RPH)])
        if RTH:
            @pl.when(sid == 0)
            def _():
                pltpu.sync_copy(acc.at[pl.ds(RPH * NS, RTH)],
                                out_hbm.at[cid].at[pl.ds(RPH * NS, RTH)])

    def _final_body(p_ref, h_ref, dinv_ref, wt_ref, o_ref):
        a = (p_ref[0] + p_ref[1] - h_ref[...]) * dinv_ref[...]
        o_ref[...] = lax.dot_general(
            a, wt_ref[...], (((1,), (0,)), ((), ())),
            preferred_element_type=jnp.float32)

    final_call = pl.pallas_call(
        _final_body,
        out_shape=jax.ShapeDtypeStruct((N, C), jnp.float32),
    )

    return deg_kernel, scale_call, scat_kernel, final_call, E_pad, EPT

def kernel(x, edge_index, W):
    N, C = x.shape
    E = edge_index.shape[1]
    deg_kernel, scale_call, scat_kernel, final_call, E_pad, EPT = \
        _build(N, E, C)

    row = edge_index[0]
    col = edge_index[1]
    pad = E_pad - E
    if pad:
        pad_iota = jnp.arange(pad, dtype=jnp.int32)
        row = jnp.concatenate([row, N + (pad_iota % TRASH)])
        col = jnp.concatenate([col, pad_iota % TRASH])
    rowd = row.reshape(NW, EPT // CHUNK, CHUNK)
    cold = col.reshape(NW, EPT // CHUNK, CHUNK)

    parts = deg_kernel(rowd)
    partsT = parts.T[:N]
    h, dinv = scale_call(partsT, x)
    p = scat_kernel(h, cold, rowd)
    return final_call(p, h, dinv, W.T)


# speedup vs baseline: 10.2439x; 1.1834x over previous
import functools

import jax
import jax.numpy as jnp
from jax import lax
from jax.experimental import pallas as pl
from jax.experimental.pallas import tpu as pltpu
from jax.experimental.pallas import tpu_sc as plsc

NC, NS, L = 2, 16, 16
NW = NC * NS
CHUNK = 128
SCH_EDGES = 64
TRASH = 16


def _mesh():
    return plsc.VectorSubcoreMesh(
        core_axis_name="c", subcore_axis_name="s",
        num_cores=NC, num_subcores=NS)


@functools.lru_cache(maxsize=None)
def _build(N, E, C):
    EPT = -(-E // (NW * CHUNK * 4)) * CHUNK * 4
    E_pad = NW * EPT
    CPT = EPT // CHUNK
    NBA = -(-(N + TRASH) // (NS * CHUNK)) * NS * CHUNK
    DPT = NBA // NS

    NBH = N + TRASH
    RPH = (N // NS) // 8 * 8
    RTH = N - RPH * NS
    SCH = SCH_EDGES
    CPS = EPT // SCH
    WIN = 40
    assert CPS % WIN == 0 and WIN % 4 == 0 and WIN >= 8

    @functools.partial(
        pl.kernel, mesh=_mesh(),
        out_type=jax.ShapeDtypeStruct((NC, NBA), jnp.float32),
        scratch_types=[
            pltpu.VMEM((CPT, CHUNK), jnp.int32),
            pltpu.VMEM((CHUNK,), jnp.float32),
            pltpu.VMEM_SHARED((NBA,), jnp.float32),
            pltpu.SemaphoreType.DMA,
        ],
    )
    def deg_kernel(row3_hbm, out_hbm, idx_v, stage_v, sdeg, sem):
        cid = lax.axis_index("c")
        sid = lax.axis_index("s")
        wid = sid * NC + cid
        pltpu.sync_copy(row3_hbm.at[wid], idx_v)

        def fill(val):
            def fb(i, _):
                stage_v[pl.ds(i * L, L)] = jnp.full((L,), val, jnp.float32)
                return 0
            lax.fori_loop(0, CHUNK // L, fb, 0)

        fill(0.0)
        z0 = sid * DPT
        for j in range(DPT // CHUNK):
            pltpu.sync_copy(stage_v, sdeg.at[pl.ds(z0 + j * CHUNK, CHUNK)])
        plsc.subcore_barrier()
        fill(1.0)

        def hb(g, _):
            c = g * 4
            s0 = pltpu.async_copy(stage_v, sdeg.at[idx_v.at[c]], sem, add=True)
            s1 = pltpu.async_copy(stage_v, sdeg.at[idx_v.at[c + 1]], sem, add=True)
            s2 = pltpu.async_copy(stage_v, sdeg.at[idx_v.at[c + 2]], sem, add=True)
            s3 = pltpu.async_copy(stage_v, sdeg.at[idx_v.at[c + 3]], sem, add=True)
            s0.wait(); s1.wait(); s2.wait(); s3.wait()
            return 0
        lax.fori_loop(0, CPT // 4, hb, 0)

        plsc.subcore_barrier()
        pltpu.sync_copy(sdeg.at[pl.ds(z0, DPT)],
                        out_hbm.at[cid].at[pl.ds(z0, DPT)])

    def _scale_body(ptsT_ref, x_ref, h_ref, dinv_ref):
        deg = jnp.sum(ptsT_ref[...], axis=1, keepdims=True)
        dinv = lax.rsqrt(deg + 1.0)
        dinv_ref[...] = dinv
        h_ref[...] = x_ref[...] * dinv

    scale_call = pl.pallas_call(
        _scale_body,
        out_shape=(jax.ShapeDtypeStruct((N, C), jnp.float32),
                   jax.ShapeDtypeStruct((N, 1), jnp.float32)),
    )

    @functools.partial(
        pl.kernel, mesh=_mesh(),
        out_type=jax.ShapeDtypeStruct((NC, N, C), jnp.float32),
        scratch_types=[
            pltpu.VMEM((WIN, SCH), jnp.int32),
            pltpu.VMEM((WIN, SCH), jnp.int32),
            pltpu.VMEM((SCH, C), jnp.float32),
            pltpu.VMEM((SCH, C), jnp.float32),
            pltpu.VMEM((SCH, C), jnp.float32),
            pltpu.VMEM((SCH, C), jnp.float32),
            pltpu.VMEM_SHARED((NBH, C), jnp.float32),
            pltpu.SemaphoreType.DMA,
            pltpu.SemaphoreType.DMA,
            pltpu.SemaphoreType.DMA,
            pltpu.SemaphoreType.DMA,
            pltpu.SemaphoreType.DMA,
            pltpu.SemaphoreType.DMA,
            pltpu.SemaphoreType.DMA,
            pltpu.SemaphoreType.DMA,
        ],
    )
    def scat_kernel(h_hbm, col3_hbm, row3_hbm, out_hbm,
                    col_v, row_v, b0, b1, b2, b3, acc,
                    gs0, gs1, gs2, gs3, ss0, ss1, ss2, ss3):
        cid = lax.axis_index("c")
        sid = lax.axis_index("s")
        wid = sid * NC + cid
        buf = (b0, b1, b2, b3)
        gsem = (gs0, gs1, gs2, gs3)
        ssem = (ss0, ss1, ss2, ss3)

        r0 = sid * RPH
        pltpu.sync_copy(h_hbm.at[pl.ds(r0, RPH)], acc.at[pl.ds(r0, RPH)])
        if RTH:
            @pl.when(sid == 0)
            def _():
                pltpu.sync_copy(h_hbm.at[pl.ds(RPH * NS, RTH)],
                                acc.at[pl.ds(RPH * NS, RTH)])
        plsc.subcore_barrier()

        def drain(sem, k):
            pltpu.make_async_copy(h_hbm.at[pl.ds(0, SCH)], buf[k], sem).wait()

        def gath(k, lc):
            pltpu.async_copy(h_hbm.at[col_v.at[lc]], buf[k], gsem[k])

        def scat(k, lc):
            pltpu.async_copy(buf[k], acc.at[row_v.at[lc]], ssem[k], add=True)

        def mbody(m, _):
            a = m * 4
            drain(gsem[2], 2); scat(2, a + 2)
            drain(gsem[3], 3); scat(3, a + 3)
            drain(ssem[0], 0); gath(0, a + 4)
            drain(ssem[1], 1); gath(1, a + 5)
            drain(gsem[0], 0); scat(0, a + 4)
            drain(gsem[1], 1); scat(1, a + 5)
            drain(ssem[2], 2); gath(2, a + 6)
            drain(ssem[3], 3); gath(3, a + 7)
            return 0

        for w in range(CPS // WIN):
            pltpu.sync_copy(col3_hbm.at[wid].at[pl.ds(w * WIN, WIN)], col_v)
            pltpu.sync_copy(row3_hbm.at[wid].at[pl.ds(w * WIN, WIN)], row_v)
            gath(0, 0)
            gath(1, 1)
            drain(gsem[0], 0); scat(0, 0)
            drain(gsem[1], 1); scat(1, 1)
            gath(2, 2)
            gath(3, 3)
            lax.fori_loop(0, (WIN - 8) // 4 + 1, mbody, 0)
            drain(gsem[2], 2); scat(2, WIN - 2)
            drain(gsem[3], 3); scat(3, WIN - 1)
            for k in range(4):
                drain(ssem[k], k)

        plsc.subcore_barrier()
        pltpu.sync_copy(acc.at[pl.ds(r0, RPH)],
                        out_hbm.at[cid].at[pl.ds(r0, RPH)])
        if RTH:
            @pl.when(sid == 0)
            def _():
                pltpu.sync_copy(acc.at[pl.ds(RPH * NS, RTH)],
                                out_hbm.at[cid].at[pl.ds(RPH * NS, RTH)])

    def _final_body(p_ref, h_ref, dinv_ref, wt_ref, o_ref):
        a = (p_ref[0] + p_ref[1] - h_ref[...]) * dinv_ref[...]
        o_ref[...] = lax.dot_general(
            a, wt_ref[...], (((1,), (0,)), ((), ())),
            preferred_element_type=jnp.float32)

    final_call = pl.pallas_call(
        _final_body,
        out_shape=jax.ShapeDtypeStruct((N, C), jnp.float32),
    )

    return deg_kernel, scale_call, scat_kernel, final_call, E_pad, EPT

def kernel(x, edge_index, W):
    N, C = x.shape
    E = edge_index.shape[1]
    deg_kernel, scale_call, scat_kernel, final_call, E_pad, EPT = \
        _build(N, E, C)

    row = edge_index[0]
    col = edge_index[1]
    pad = E_pad - E
    if pad:
        pad_iota = jnp.arange(pad, dtype=jnp.int32)
        row = jnp.concatenate([row, N + (pad_iota % TRASH)])
        col = jnp.concatenate([col, pad_iota % TRASH])
    rowd = row.reshape(NW, EPT // CHUNK, CHUNK)
    rows = row.reshape(NW, EPT // SCH_EDGES, SCH_EDGES)
    cols = col.reshape(NW, EPT // SCH_EDGES, SCH_EDGES)

    parts = deg_kernel(rowd)
    partsT = parts.T[:N]
    h, dinv = scale_call(partsT, x)
    p = scat_kernel(h, cols, rows)
    return final_call(p, h, dinv, W.T)
